# Initial kernel scaffold; baseline (speedup 1.0000x reference)
import jax
import jax.numpy as jnp
from jax.experimental import pallas as pl


def kernel(x, edge_index, batch, W0, b0, W1, b1, W2, b2, W3, b3, Wl, bl):
    raise NotImplementedError("write your pallas kernel here")


# speedup vs baseline: 13.4586x; 13.4586x over previous
import functools

import jax
import jax.numpy as jnp
from jax import lax
from jax.experimental import pallas as pl
from jax.experimental.pallas import tpu as pltpu
from jax.experimental.pallas import tpu_sc as plsc

N = 10000
E = 160000
G = 64
D_IN = 256
D_H = 512
NEG_SLOPE = 0.01

NS = 16
K = 80
EPT = E // NS
NB = EPT // K
TR = 632
TR_LAST = N - (NS - 1) * TR


def _leaky(v):
    return jnp.where(v >= 0, v, NEG_SLOPE * v)


@functools.lru_cache(maxsize=None)
def _make_sc_agg(C, CH, gather=True):
    mesh = plsc.VectorSubcoreMesh(core_axis_name="c", subcore_axis_name="s",
                                  num_cores=2, num_subcores=NS)

    scratch = [
        pltpu.VMEM((EPT,), jnp.int32),
        pltpu.VMEM((NB, K), jnp.int32),
        pltpu.VMEM((K,), jnp.int32),
        pltpu.VMEM((K,), jnp.int32),
        pltpu.VMEM((K, CH), jnp.float32),
        pltpu.VMEM((K, CH), jnp.float32),
        pltpu.VMEM_SHARED((N, CH), jnp.float32),
        pltpu.SemaphoreType.DMA,
        pltpu.SemaphoreType.DMA,
    ]

    @functools.partial(
        pl.kernel,
        out_type=jax.ShapeDtypeStruct((C, N, CH), jnp.float32),
        mesh=mesh,
        scratch_types=scratch,
    )
    def agg(hp_hbm, src_hbm, dst3_hbm, zeros_hbm, out_hbm,
            src_v, dst_v, idx0, idx1, st0, st1, acc, s0, s1):
        c = lax.axis_index("c")
        s = lax.axis_index("s")
        rbase = pl.multiple_of(s * TR, 8)

        def fill(buf, rows, val):
            per_row = CH // 16

            def zf(i, _):
                r = i // per_row
                col = (i % per_row) * 16
                buf[r, pl.ds(col, 16)] = jnp.full((16,), val, jnp.float32)
                return 0

            lax.fori_loop(0, rows * per_row, zf, 0)

        def run_core(my_chunks):
            if gather:
                pltpu.sync_copy(src_hbm.at[pl.ds(s * EPT, EPT)], src_v)
            pltpu.sync_copy(dst3_hbm.at[s], dst_v)

            if not gather:
                fill(st0, K, 1.0)

            for ch in my_chunks:
                process_chunk(ch)

        def process_chunk(ch):
            @pl.when(s < NS - 1)
            def _():
                pltpu.sync_copy(zeros_hbm.at[pl.ds(rbase, TR)],
                                acc.at[pl.ds(rbase, TR)])

            @pl.when(s == NS - 1)
            def _():
                pltpu.sync_copy(zeros_hbm.at[pl.ds((NS - 1) * TR, TR_LAST)],
                                acc.at[pl.ds((NS - 1) * TR, TR_LAST)])

            plsc.subcore_barrier()

            def g(j, idx, buf, sem):
                if C == 1:
                    for t in range(K // 16):
                        idx[pl.ds(t * 16, 16)] = (
                            src_v[pl.ds(j * K + t * 16, 16)])
                else:
                    for t in range(K // 16):
                        idx[pl.ds(t * 16, 16)] = (
                            src_v[pl.ds(j * K + t * 16, 16)] * C + ch)
                return pltpu.async_copy(hp_hbm.at[idx], buf, sem)

            def wg(idx, buf, sem):
                pltpu.make_async_copy(hp_hbm.at[idx], buf, sem).wait()

            def scat(j, buf):
                pltpu.sync_copy(buf, acc.at[dst_v.at[j]], add=True)

            if gather:
                g(0, idx0, st0, s0)

                def lp(i, _):
                    g(2 * i + 1, idx1, st1, s1)
                    wg(idx0, st0, s0)
                    scat(2 * i, st0)
                    g(2 * i + 2, idx0, st0, s0)
                    wg(idx1, st1, s1)
                    scat(2 * i + 1, st1)
                    return 0

                lax.fori_loop(0, (NB - 1) // 2, lp, 0)
                wg(idx0, st0, s0)
                scat(NB - 1, st0)
            else:
                def lp1(j, _):
                    scat(j, st0)
                    return 0

                lax.fori_loop(0, NB, lp1, 0)

            plsc.subcore_barrier()

            @pl.when(s < NS - 1)
            def _():
                pltpu.sync_copy(acc.at[pl.ds(rbase, TR)],
                                out_hbm.at[ch, pl.ds(rbase, TR)])

            @pl.when(s == NS - 1)
            def _():
                pltpu.sync_copy(
                    acc.at[pl.ds((NS - 1) * TR, TR_LAST)],
                    out_hbm.at[ch, pl.ds((NS - 1) * TR, TR_LAST)])

        chunks0 = list(range(0, C, 2))
        chunks1 = list(range(1, C, 2))

        @pl.when(c == 0)
        def _():
            run_core(chunks0)

        if chunks1:
            @pl.when(c == 1)
            def _():
                run_core(chunks1)

    return agg


def _sc_deg(dst2):
    dummy_hp = jnp.zeros((8, 16), jnp.float32)
    dummy_src = jnp.zeros((E,), jnp.int32)
    z = jnp.zeros((N, 16), jnp.float32)
    return _make_sc_agg(1, 16, gather=False)(dummy_hp, dummy_src, dst2, z)[0]


def _sc_agg_c1(hp, src, dst2):
    return _make_sc_agg(1, 128)(hp, src, dst2, jnp.zeros((N, 128), jnp.float32))


def _sc_agg_c2(hp, src, dst2):
    return _make_sc_agg(2, 128)(hp, src, dst2, jnp.zeros((N, 128), jnp.float32))


def _sc_agg_c4(hp, src, dst2):
    return _make_sc_agg(4, 128)(hp, src, dst2, jnp.zeros((N, 128), jnp.float32))


BM = 1000
MB = N // BM


def _t_pre_body(deg_ref, x_ref, dinv_ref, xp_ref):
    deg = deg_ref[:, 0:1] + 1.0
    dinv = lax.rsqrt(deg)
    dinv_ref[...] = dinv
    xp_ref[...] = x_ref[...] * dinv


def _t_pre(deg16, x):
    return pl.pallas_call(
        _t_pre_body,
        grid=(MB,),
        in_specs=[
            pl.BlockSpec((BM, 16), lambda m: (m, 0)),
            pl.BlockSpec((BM, D_IN), lambda m: (m, 0)),
        ],
        out_specs=[
            pl.BlockSpec((BM, 1), lambda m: (m, 0)),
            pl.BlockSpec((BM, D_IN), lambda m: (m, 0)),
        ],
        out_shape=[
            jax.ShapeDtypeStruct((N, 1), jnp.float32),
            jax.ShapeDtypeStruct((N, D_IN), jnp.float32),
        ],
    )(deg16, x)


def _t0_body(agg_ref, xp_ref, dinv_ref, W0_ref, W1_ref, b0_ref, out_ref, acc):
    k = pl.program_id(1)
    dinv = dinv_ref[...]
    z = dinv * (agg_ref[0] + xp_ref[...])
    part = jnp.dot(z, W0_ref[...], preferred_element_type=jnp.float32)

    @pl.when(k == 0)
    def _():
        acc[...] = part

    @pl.when(k > 0)
    def _():
        acc[...] += part

    @pl.when(k == pl.num_programs(1) - 1)
    def _():
        h0 = _leaky(acc[...] + b0_ref[...])
        out_ref[...] = dinv * jnp.dot(
            h0, W1_ref[...], preferred_element_type=jnp.float32)


def _t0(aggx, xp, dinv, W0, W1, b0):
    C = D_IN // 128
    return pl.pallas_call(
        _t0_body,
        grid=(MB, C),
        in_specs=[
            pl.BlockSpec((1, BM, 128), lambda m, k: (k, m, 0)),
            pl.BlockSpec((BM, 128), lambda m, k: (m, k)),
            pl.BlockSpec((BM, 1), lambda m, k: (m, 0)),
            pl.BlockSpec((128, D_H), lambda m, k: (k, 0)),
            pl.BlockSpec((D_H, D_H), lambda m, k: (0, 0)),
            pl.BlockSpec((1, D_H), lambda m, k: (0, 0)),
        ],
        out_specs=pl.BlockSpec((BM, D_H), lambda m, k: (m, 0)),
        out_shape=jax.ShapeDtypeStruct((N, D_H), jnp.float32),
        scratch_shapes=[pltpu.VMEM((BM, D_H), jnp.float32)],
        compiler_params=pltpu.CompilerParams(
            dimension_semantics=("parallel", "arbitrary")),
    )(aggx, xp, dinv, W0, W1, b0)


def _t_mid_body(agg_ref, up_ref, dinv_ref, b_ref, W_ref, out_ref, acc):
    k = pl.program_id(1)
    dinv = dinv_ref[...]
    hc = _leaky(dinv * (agg_ref[0] + up_ref[...]) + b_ref[0])
    part = jnp.dot(hc, W_ref[...], preferred_element_type=jnp.float32)

    @pl.when(k == 0)
    def _():
        acc[...] = part

    @pl.when(k > 0)
    def _():
        acc[...] += part

    @pl.when(k == pl.num_programs(1) - 1)
    def _():
        out_ref[...] = dinv * acc[...]


def _t_mid(agg, up, dinv, b, W, d_out):
    C = D_H // 128
    return pl.pallas_call(
        _t_mid_body,
        grid=(MB, C),
        in_specs=[
            pl.BlockSpec((1, BM, 128), lambda m, k: (k, m, 0)),
            pl.BlockSpec((BM, 128), lambda m, k: (m, k)),
            pl.BlockSpec((BM, 1), lambda m, k: (m, 0)),
            pl.BlockSpec((1, 1, 128), lambda m, k: (k, 0, 0)),
            pl.BlockSpec((128, d_out), lambda m, k: (k, 0)),
        ],
        out_specs=pl.BlockSpec((BM, d_out), lambda m, k: (m, 0)),
        out_shape=jax.ShapeDtypeStruct((N, d_out), jnp.float32),
        scratch_shapes=[pltpu.VMEM((BM, d_out), jnp.float32)],
        compiler_params=pltpu.CompilerParams(
            dimension_semantics=("parallel", "arbitrary")),
    )(agg, up, dinv, b, W)


def _t3_body(agg_ref, up_ref, dinv_ref, b3_ref, batch_ref, Wl_ref, bl_ref,
             out_ref):
    dinv = dinv_ref[...]
    h3 = _leaky(dinv * (agg_ref[0][:, :16] + up_ref[:, :16])
                + b3_ref[...])
    gids = lax.broadcasted_iota(jnp.int32, (G, N), 0)
    onehot = (gids == batch_ref[...]).astype(jnp.float32)
    sums = jnp.dot(onehot, h3, preferred_element_type=jnp.float32)
    cnt = jnp.sum(onehot, axis=1, keepdims=True)
    g = sums / jnp.maximum(cnt, 1.0)
    logits = jnp.dot(g, Wl_ref[...], preferred_element_type=jnp.float32)
    logits = logits + bl_ref[...]
    col = lax.broadcasted_iota(jnp.int32, (G, 16), 1)
    logits = jnp.where(col < 10, logits, -1e30)
    m = jnp.max(logits, axis=1, keepdims=True)
    lse = jnp.log(jnp.sum(jnp.exp(logits - m), axis=1, keepdims=True))
    out_ref[...] = logits - m - lse


def _t3(agg3, up3, dinv, b3p, batch2, Wlp, blp):
    return pl.pallas_call(
        _t3_body,
        out_shape=jax.ShapeDtypeStruct((G, 16), jnp.float32),
    )(agg3, up3, dinv, b3p, batch2, Wlp, blp)


def kernel(x, edge_index, batch, W0, b0, W1, b1, W2, b2, W3, b3, Wl, bl):
    src = edge_index[0]
    dst2 = edge_index[1].reshape(NS, NB, K)

    W3p = jnp.zeros((D_H, 128), jnp.float32).at[:, :10].set(W3)
    b3p = jnp.zeros((1, 16), jnp.float32).at[0, :10].set(b3)
    Wlp = jnp.zeros((16, 16), jnp.float32).at[:10, :10].set(Wl)
    blp = jnp.zeros((1, 16), jnp.float32).at[0, :10].set(bl)

    deg16 = _sc_deg(dst2)

    dinv, xp = _t_pre(deg16, x)

    aggx = _sc_agg_c2(xp.reshape(N * 2, 128), src, dst2)
    up1 = _t0(aggx, xp, dinv, W0, W1, b0.reshape(1, D_H))

    agg1 = _sc_agg_c4(up1.reshape(N * 4, 128), src, dst2)
    up2 = _t_mid(agg1, up1, dinv, b1.reshape(4, 1, 128), W2, D_H)

    agg2 = _sc_agg_c4(up2.reshape(N * 4, 128), src, dst2)
    up3 = _t_mid(agg2, up2, dinv, b2.reshape(4, 1, 128), W3p, 128)

    agg3 = _sc_agg_c1(up3, src, dst2)

    out16 = _t3(agg3, up3, dinv, b3p, batch.reshape(1, N), Wlp, blp)
    return out16[:, :10]


# speedup vs baseline: 14.0018x; 1.0404x over previous
import functools

import jax
import jax.numpy as jnp
from jax import lax
from jax.experimental import pallas as pl
from jax.experimental.pallas import tpu as pltpu
from jax.experimental.pallas import tpu_sc as plsc

N = 10000
E = 160000
G = 64
D_IN = 256
D_H = 512
NEG_SLOPE = 0.01

NS = 16
K = 80
EPT = E // NS
NB = EPT // K
TR = 632
TR_LAST = N - (NS - 1) * TR


def _leaky(v):
    return jnp.where(v >= 0, v, NEG_SLOPE * v)


@functools.lru_cache(maxsize=None)
def _make_sc_agg(C, CH, gather=True, edge_split=False):
    mesh = plsc.VectorSubcoreMesh(core_axis_name="c", subcore_axis_name="s",
                                  num_cores=2, num_subcores=NS)
    n_out = 2 if edge_split else C

    scratch = [
        pltpu.VMEM((EPT,), jnp.int32),
        pltpu.VMEM((NB, K), jnp.int32),
        pltpu.VMEM((K,), jnp.int32),
        pltpu.VMEM((K,), jnp.int32),
        pltpu.VMEM((K, CH), jnp.float32),
        pltpu.VMEM((K, CH), jnp.float32),
        pltpu.VMEM_SHARED((N, CH), jnp.float32),
        pltpu.SemaphoreType.DMA,
        pltpu.SemaphoreType.DMA,
    ]

    @functools.partial(
        pl.kernel,
        out_type=jax.ShapeDtypeStruct((n_out, N, CH), jnp.float32),
        mesh=mesh,
        scratch_types=scratch,
    )
    def agg(hp_hbm, src_hbm, dst3_hbm, zeros_hbm, out_hbm,
            src_v, dst_v, idx0, idx1, st0, st1, acc, s0, s1):
        c = lax.axis_index("c")
        s = lax.axis_index("s")
        rbase = pl.multiple_of(s * TR, 8)

        def fill(buf, rows, val):
            per_row = CH // 16

            def zf(i, _):
                r = i // per_row
                col = (i % per_row) * 16
                buf[r, pl.ds(col, 16)] = jnp.full((16,), val, jnp.float32)
                return 0

            lax.fori_loop(0, rows * per_row, zf, 0)

        def run_core(my_chunks, jlo=0, jn=NB, out_base=None):
            if gather:
                pltpu.sync_copy(src_hbm.at[pl.ds(s * EPT, EPT)], src_v)
            pltpu.sync_copy(dst3_hbm.at[s], dst_v)

            if not gather:
                fill(st0, K, 1.0)

            for ch in my_chunks:
                process_chunk(ch, ch if out_base is None else out_base,
                              jlo, jn)

        def process_chunk(ch, out_slot, jlo, jn):
            @pl.when(s < NS - 1)
            def _():
                pltpu.sync_copy(zeros_hbm.at[pl.ds(rbase, TR)],
                                acc.at[pl.ds(rbase, TR)])

            @pl.when(s == NS - 1)
            def _():
                pltpu.sync_copy(zeros_hbm.at[pl.ds((NS - 1) * TR, TR_LAST)],
                                acc.at[pl.ds((NS - 1) * TR, TR_LAST)])

            plsc.subcore_barrier()

            def g(j, idx, buf, sem):
                if C == 1 and not edge_split:
                    for t in range(K // 16):
                        idx[pl.ds(t * 16, 16)] = (
                            src_v[pl.ds(j * K + t * 16, 16)])
                else:
                    for t in range(K // 16):
                        idx[pl.ds(t * 16, 16)] = (
                            src_v[pl.ds(j * K + t * 16, 16)]
                            * jnp.int32(C) + jnp.int32(ch))
                return pltpu.async_copy(hp_hbm.at[idx], buf, sem)

            def wg(idx, buf, sem):
                pltpu.make_async_copy(hp_hbm.at[idx], buf, sem).wait()

            def scat(j, buf):
                pltpu.sync_copy(buf, acc.at[dst_v.at[j]], add=True)

            if gather:
                g(jlo, idx0, st0, s0)

                def lp(i, _):
                    j = jlo + 2 * i
                    g(j + 1, idx1, st1, s1)
                    wg(idx0, st0, s0)
                    scat(j, st0)
                    g(j + 2, idx0, st0, s0)
                    wg(idx1, st1, s1)
                    scat(j + 1, st1)
                    return 0

                lax.fori_loop(0, (jn - 1) // 2, lp, 0)
                if jn % 2 == 1:
                    wg(idx0, st0, s0)
                    scat(jlo + jn - 1, st0)
                else:
                    g(jlo + jn - 1, idx1, st1, s1)
                    wg(idx0, st0, s0)
                    scat(jlo + jn - 2, st0)
                    wg(idx1, st1, s1)
                    scat(jlo + jn - 1, st1)
            else:
                def lp1(i, _):
                    scat(jlo + i, st0)
                    return 0

                lax.fori_loop(0, jn, lp1, 0)

            plsc.subcore_barrier()

            @pl.when(s < NS - 1)
            def _():
                pltpu.sync_copy(acc.at[pl.ds(rbase, TR)],
                                out_hbm.at[out_slot, pl.ds(rbase, TR)])

            @pl.when(s == NS - 1)
            def _():
                pltpu.sync_copy(
                    acc.at[pl.ds((NS - 1) * TR, TR_LAST)],
                    out_hbm.at[out_slot, pl.ds((NS - 1) * TR, TR_LAST)])

        if edge_split:
            assert C == 1
            nb0 = (NB + 1) // 2

            @pl.when(c == 0)
            def _():
                run_core([0], jlo=0, jn=nb0, out_base=0)

            @pl.when(c == 1)
            def _():
                run_core([0], jlo=nb0, jn=NB - nb0, out_base=1)
        else:
            chunks0 = list(range(0, C, 2))
            chunks1 = list(range(1, C, 2))

            @pl.when(c == 0)
            def _():
                run_core(chunks0)

            if chunks1:
                @pl.when(c == 1)
                def _():
                    run_core(chunks1)

    return agg


def _sc_deg(dst2):
    dummy_hp = jnp.zeros((8, 16), jnp.float32)
    dummy_src = jnp.zeros((E,), jnp.int32)
    z = jnp.zeros((N, 16), jnp.float32)
    out = _make_sc_agg(1, 16, gather=False)(dummy_hp, dummy_src, dst2, z)
    return jnp.concatenate([out, jnp.zeros_like(out)], axis=0)


def _sc_agg_c1(hp, src, dst2):
    return _make_sc_agg(1, 128, edge_split=True)(
        hp, src, dst2, jnp.zeros((N, 128), jnp.float32))


def _sc_agg_c2(hp, src, dst2):
    return _make_sc_agg(2, 128)(hp, src, dst2, jnp.zeros((N, 128), jnp.float32))


def _sc_agg_c4(hp, src, dst2):
    return _make_sc_agg(4, 128)(hp, src, dst2, jnp.zeros((N, 128), jnp.float32))


BM = 1000
MB = N // BM


def _t_pre_body(deg_ref, x_ref, dinv_ref, xp_ref):
    deg = deg_ref[0, :, 0:1] + deg_ref[1, :, 0:1] + 1.0
    dinv = lax.rsqrt(deg)
    dinv_ref[...] = dinv
    xp_ref[...] = x_ref[...] * dinv


def _t_pre(deg16, x):
    return pl.pallas_call(
        _t_pre_body,
        grid=(MB,),
        in_specs=[
            pl.BlockSpec((2, BM, 16), lambda m: (0, m, 0)),
            pl.BlockSpec((BM, D_IN), lambda m: (m, 0)),
        ],
        out_specs=[
            pl.BlockSpec((BM, 1), lambda m: (m, 0)),
            pl.BlockSpec((BM, D_IN), lambda m: (m, 0)),
        ],
        out_shape=[
            jax.ShapeDtypeStruct((N, 1), jnp.float32),
            jax.ShapeDtypeStruct((N, D_IN), jnp.float32),
        ],
    )(deg16, x)


def _t0_body(agg_ref, xp_ref, dinv_ref, W0_ref, W1_ref, b0_ref, out_ref, acc):
    k = pl.program_id(1)
    dinv = dinv_ref[...]
    z = dinv * (agg_ref[0] + xp_ref[...])
    part = jnp.dot(z, W0_ref[...], preferred_element_type=jnp.float32)

    @pl.when(k == 0)
    def _():
        acc[...] = part

    @pl.when(k > 0)
    def _():
        acc[...] += part

    @pl.when(k == pl.num_programs(1) - 1)
    def _():
        h0 = _leaky(acc[...] + b0_ref[...])
        out_ref[...] = dinv * jnp.dot(
            h0, W1_ref[...], preferred_element_type=jnp.float32)


def _t0(aggx, xp, dinv, W0, W1, b0):
    C = D_IN // 128
    return pl.pallas_call(
        _t0_body,
        grid=(MB, C),
        in_specs=[
            pl.BlockSpec((1, BM, 128), lambda m, k: (k, m, 0)),
            pl.BlockSpec((BM, 128), lambda m, k: (m, k)),
            pl.BlockSpec((BM, 1), lambda m, k: (m, 0)),
            pl.BlockSpec((128, D_H), lambda m, k: (k, 0)),
            pl.BlockSpec((D_H, D_H), lambda m, k: (0, 0)),
            pl.BlockSpec((1, D_H), lambda m, k: (0, 0)),
        ],
        out_specs=pl.BlockSpec((BM, D_H), lambda m, k: (m, 0)),
        out_shape=jax.ShapeDtypeStruct((N, D_H), jnp.float32),
        scratch_shapes=[pltpu.VMEM((BM, D_H), jnp.float32)],
        compiler_params=pltpu.CompilerParams(
            dimension_semantics=("parallel", "arbitrary")),
    )(aggx, xp, dinv, W0, W1, b0)


def _t_mid_body(agg_ref, up_ref, dinv_ref, b_ref, W_ref, out_ref, acc):
    k = pl.program_id(1)
    dinv = dinv_ref[...]
    hc = _leaky(dinv * (agg_ref[0] + up_ref[...]) + b_ref[0])
    part = jnp.dot(hc, W_ref[...], preferred_element_type=jnp.float32)

    @pl.when(k == 0)
    def _():
        acc[...] = part

    @pl.when(k > 0)
    def _():
        acc[...] += part

    @pl.when(k == pl.num_programs(1) - 1)
    def _():
        out_ref[...] = dinv * acc[...]


def _t_mid(agg, up, dinv, b, W, d_out):
    C = D_H // 128
    return pl.pallas_call(
        _t_mid_body,
        grid=(MB, C),
        in_specs=[
            pl.BlockSpec((1, BM, 128), lambda m, k: (k, m, 0)),
            pl.BlockSpec((BM, 128), lambda m, k: (m, k)),
            pl.BlockSpec((BM, 1), lambda m, k: (m, 0)),
            pl.BlockSpec((1, 1, 128), lambda m, k: (k, 0, 0)),
            pl.BlockSpec((128, d_out), lambda m, k: (k, 0)),
        ],
        out_specs=pl.BlockSpec((BM, d_out), lambda m, k: (m, 0)),
        out_shape=jax.ShapeDtypeStruct((N, d_out), jnp.float32),
        scratch_shapes=[pltpu.VMEM((BM, d_out), jnp.float32)],
        compiler_params=pltpu.CompilerParams(
            dimension_semantics=("parallel", "arbitrary")),
    )(agg, up, dinv, b, W)


def _t3_body(agg_ref, up_ref, dinv_ref, b3_ref, batch_ref, Wl_ref, bl_ref,
             out_ref):
    dinv = dinv_ref[...]
    h3 = _leaky(dinv * (agg_ref[0][:, :16] + agg_ref[1][:, :16]
                        + up_ref[:, :16])
                + b3_ref[...])
    gids = lax.broadcasted_iota(jnp.int32, (G, N), 0)
    onehot = (gids == batch_ref[...]).astype(jnp.float32)
    sums = jnp.dot(onehot, h3, preferred_element_type=jnp.float32)
    cnt = jnp.sum(onehot, axis=1, keepdims=True)
    g = sums / jnp.maximum(cnt, 1.0)
    logits = jnp.dot(g, Wl_ref[...], preferred_element_type=jnp.float32)
    logits = logits + bl_ref[...]
    col = lax.broadcasted_iota(jnp.int32, (G, 16), 1)
    logits = jnp.where(col < 10, logits, -1e30)
    m = jnp.max(logits, axis=1, keepdims=True)
    lse = jnp.log(jnp.sum(jnp.exp(logits - m), axis=1, keepdims=True))
    out_ref[...] = logits - m - lse


def _t3(agg3, up3, dinv, b3p, batch2, Wlp, blp):
    return pl.pallas_call(
        _t3_body,
        out_shape=jax.ShapeDtypeStruct((G, 16), jnp.float32),
    )(agg3, up3, dinv, b3p, batch2, Wlp, blp)


def kernel(x, edge_index, batch, W0, b0, W1, b1, W2, b2, W3, b3, Wl, bl):
    src = edge_index[0]
    dst2 = edge_index[1].reshape(NS, NB, K)

    W3p = jnp.zeros((D_H, 128), jnp.float32).at[:, :10].set(W3)
    b3p = jnp.zeros((1, 16), jnp.float32).at[0, :10].set(b3)
    Wlp = jnp.zeros((16, 16), jnp.float32).at[:10, :10].set(Wl)
    blp = jnp.zeros((1, 16), jnp.float32).at[0, :10].set(bl)

    deg16 = _sc_deg(dst2)

    dinv, xp = _t_pre(deg16, x)

    aggx = _sc_agg_c2(xp.reshape(N * 2, 128), src, dst2)
    up1 = _t0(aggx, xp, dinv, W0, W1, b0.reshape(1, D_H))

    agg1 = _sc_agg_c4(up1.reshape(N * 4, 128), src, dst2)
    up2 = _t_mid(agg1, up1, dinv, b1.reshape(4, 1, 128), W2, D_H)

    agg2 = _sc_agg_c4(up2.reshape(N * 4, 128), src, dst2)
    up3 = _t_mid(agg2, up2, dinv, b2.reshape(4, 1, 128), W3p, 128)

    agg3 = _sc_agg_c1(up3, src, dst2)

    out16 = _t3(agg3, up3, dinv, b3p, batch.reshape(1, N), Wlp, blp)
    return out16[:, :10]
